# baseline (speedup 1.0000x reference)
import jax
import jax.numpy as jnp
from jax.experimental import pallas as pl

_N = 10000
_F_IN = 128
_F_OUT = 32
_NUM_CLASSES = 10
_P_ROWS = _F_IN + 8 + _F_OUT


def _fused_gcn_cell(x_ref, p_ref, o_ref):
    comb = p_ref[:_F_IN, :]
    bcat = p_ref[_F_IN:_F_IN + 1, :]
    blv = p_ref[_F_IN + 1:_F_IN + 2, :_NUM_CLASSES]
    wl = p_ref[_F_IN + 8:_P_ROWS, :_NUM_CLASSES]

    g = jnp.dot(x_ref[...], comb, preferred_element_type=jnp.float32,
                precision=jax.lax.Precision.DEFAULT) + bcat
    t = jnp.tanh(g)
    one_minus_z = 1.0 + t[:, :_F_OUT]
    ht = t[:, _F_OUT:]
    h = jax.nn.relu(one_minus_z * ht)
    o_ref[...] = (
        jnp.dot(h, wl, preferred_element_type=jnp.float32) + blv)


def kernel(x, edge_index, edge_weight, Wz, bz, Wr, br, Wh, bh, Wl, bl):
    del edge_index, edge_weight, Wr, br
    az = (Wz[0, 0, :_F_IN, :] + Wz[1, 0, :_F_IN, :]) * -0.5
    ah = Wh[0, 0, :_F_IN, :] + Wh[1, 0, :_F_IN, :]
    comb = jnp.concatenate([az, ah], axis=1)
    brow = jnp.concatenate([bz * -0.5, bh])[None, :]
    blrow = jnp.pad(bl, (0, 2 * _F_OUT - _NUM_CLASSES))[None, :]
    pad6 = jnp.zeros((6, 2 * _F_OUT), jnp.float32)
    wlp = jnp.pad(Wl * 0.5, ((0, 0), (0, 2 * _F_OUT - _NUM_CLASSES)))
    packed = jnp.concatenate([comb, brow, blrow, pad6, wlp], axis=0)
    return pl.pallas_call(
        _fused_gcn_cell,
        out_shape=jax.ShapeDtypeStruct((_N, _NUM_CLASSES), jnp.float32),
    )(x, packed)


# speedup vs baseline: 1.0991x; 1.0991x over previous
import jax
import jax.numpy as jnp
from jax.experimental import pallas as pl

_N = 10000
_F_IN = 128
_F_OUT = 32
_NUM_CLASSES = 10


def _fused_gcn_cell(x_ref, wz_ref, bz_ref, wh_ref, bh_ref, wl_ref, bl_ref,
                    o_ref):
    az = (wz_ref[0, 0, :_F_IN, :] + wz_ref[1, 0, :_F_IN, :]) * -0.5
    ah = wh_ref[0, 0, :_F_IN, :] + wh_ref[1, 0, :_F_IN, :]
    comb = jnp.concatenate([az, ah], axis=1)
    bcat = jnp.concatenate([bz_ref[...] * -0.5, bh_ref[...]], axis=1)

    g = jnp.dot(x_ref[...], comb, preferred_element_type=jnp.float32,
                precision=jax.lax.Precision.DEFAULT) + bcat
    t = jnp.tanh(g)
    one_minus_z = 1.0 + t[:, :_F_OUT]
    ht = t[:, _F_OUT:]
    h = jax.nn.relu(one_minus_z * ht)
    o_ref[...] = (
        jnp.dot(h, wl_ref[...] * 0.5, preferred_element_type=jnp.float32)
        + bl_ref[...])


def kernel(x, edge_index, edge_weight, Wz, bz, Wr, br, Wh, bh, Wl, bl):
    del edge_index, edge_weight, Wr, br
    return pl.pallas_call(
        _fused_gcn_cell,
        out_shape=jax.ShapeDtypeStruct((_N, _NUM_CLASSES), jnp.float32),
    )(x, Wz, bz.reshape(1, _F_OUT),
      Wh, bh.reshape(1, _F_OUT), Wl, bl.reshape(1, _NUM_CLASSES))
